# baseline (speedup 1.0000x reference)
import functools

import jax
import jax.numpy as jnp
from jax import lax
from jax.experimental import pallas as pl
from jax.experimental.pallas import tpu as pltpu
from jax.experimental.pallas import tpu_sc as plsc

AXES_LIMIT = 40.0
RESOLUTION = 0.25
WL = int(AXES_LIMIT * 2 / RESOLUTION)

B = 2
C = 128
N = 20000
NPAD = 20480
HW = WL * WL

P = 512
SUB = 128
NSUBG = P // SUB
CPB = 40
TAIL = N - 39 * P
NCH = B * CPB
NTILES = 32
MAXC = (NCH + NTILES - 1) // NTILES


def _prep_kernel(locT_ref, idx_ref, msk_ref):
    x = locT_ref[0, 0]
    y = locT_ref[0, 1]
    m = (x > -1.0) & (x < 1.0) & (y > -1.0) & (y < 1.0)
    xs = jnp.clip(x, -0.999, 0.999) * AXES_LIMIT
    ys = jnp.clip(y, -0.999, 0.999) * AXES_LIMIT
    row = ((AXES_LIMIT - ys) / RESOLUTION).astype(jnp.int32)
    col = ((AXES_LIMIT + xs) / RESOLUTION).astype(jnp.int32)
    base = pl.program_id(0) * HW
    idx_ref[0, 0] = jnp.where(m, base + row * WL + col, 0)
    msk_ref[0, 0] = m.astype(jnp.int32)


_prep = pl.pallas_call(
    _prep_kernel,
    out_shape=(jax.ShapeDtypeStruct((B, 1, NPAD), jnp.int32),
               jax.ShapeDtypeStruct((B, 1, NPAD), jnp.int32)),
    grid=(B,),
    in_specs=[pl.BlockSpec((1, 2, NPAD), lambda b: (b, 0, 0))],
    out_specs=(pl.BlockSpec((1, 1, NPAD), lambda b: (b, 0, 0)),
               pl.BlockSpec((1, 1, NPAD), lambda b: (b, 0, 0))),
)


def _sc_body(table_hbm, idx_hbm, msk_hbm, empty_hbm, out_hbm,
             rowsbuf, idxbuf, emptybuf, mskbuf, sem):
    cid = lax.axis_index("c")
    sid = lax.axis_index("s")
    t = sid * 2 + cid

    pltpu.sync_copy(empty_hbm, emptybuf)
    evs = [emptybuf[pl.ds(16 * j, 16)] for j in range(8)]

    def do_chunk(b, k, npts, nsub):
        del nsub
        n0 = pl.multiple_of(k * P, SUB)
        pltpu.sync_copy(idx_hbm.at[b, 0, pl.ds(n0, npts)],
                        idxbuf.at[pl.ds(0, npts)])
        pltpu.sync_copy(msk_hbm.at[b, 0, pl.ds(n0, npts)],
                        mskbuf.at[pl.ds(0, npts)])
        copies = [
            pltpu.async_copy(
                table_hbm.at[idxbuf[pl.ds(16 * g, 16)]],
                rowsbuf.at[pl.ds(16 * g, 16), :], sem)
            for g in range(npts // 16)
        ]
        for cp in copies:
            cp.wait()

        def fix_body(i, _):
            i16 = jnp.zeros((16,), jnp.int32) + i
            mv = plsc.load_gather(mskbuf, [i16])
            pred = mv != 0
            for jj in range(8):
                r = rowsbuf[i, pl.ds(16 * jj, 16)]
                rowsbuf[i, pl.ds(16 * jj, 16)] = jnp.where(pred, r, evs[jj])
            return 0

        lax.fori_loop(0, npts, fix_body, 0)
        out0 = pl.multiple_of(b * N + k * P, 8)
        pltpu.sync_copy(rowsbuf.at[pl.ds(0, npts), :],
                        out_hbm.at[pl.ds(out0, npts), :])

    for c in range(MAXC):
        j = t + NTILES * c

        @pl.when(j < NCH)
        def _chunk():
            b = j // CPB
            k = j % CPB

            @pl.when(k < CPB - 1)
            def _full():
                do_chunk(b, k, P, NSUBG)

            @pl.when(k == CPB - 1)
            def _tail():
                do_chunk(b, k, TAIL, 1)


@functools.partial(
    pl.kernel,
    out_type=jax.ShapeDtypeStruct((B * N, C), jnp.float32),
    mesh=plsc.VectorSubcoreMesh(core_axis_name="c", subcore_axis_name="s"),
    compiler_params=pltpu.CompilerParams(needs_layout_passes=False),
    scratch_types=[
        pltpu.VMEM((P, C), jnp.float32),
        pltpu.VMEM((P,), jnp.int32),
        pltpu.VMEM((C,), jnp.float32),
        pltpu.VMEM((P,), jnp.int32),
        pltpu.SemaphoreType.DMA,
    ],
)
def _sc_gather(table_hbm, idx_hbm, msk_hbm, empty_hbm, out_hbm, *scratch):
    _sc_body(table_hbm, idx_hbm, msk_hbm, empty_hbm, out_hbm, *scratch)


def kernel(fmap, loc, empty):
    table = fmap.transpose(0, 2, 3, 1).reshape(B * HW, C)
    locT = jnp.pad(loc.transpose(0, 2, 1), ((0, 0), (0, 0), (0, NPAD - N)),
                   constant_values=5.0)
    idx, msk = _prep(locT)
    out = _sc_gather(table, idx, msk, empty)
    return out.reshape(B, N, C)


# speedup vs baseline: 15.1776x; 15.1776x over previous
import functools

import jax
import jax.numpy as jnp
from jax import lax
from jax.experimental import pallas as pl
from jax.experimental.pallas import tpu as pltpu
from jax.experimental.pallas import tpu_sc as plsc

AXES_LIMIT = 40.0
RESOLUTION = 0.25
WL = int(AXES_LIMIT * 2 / RESOLUTION)

B = 2
C = 128
N = 20000
NPAD = 20480
HW = WL * WL

P = 512
SUB = 128
NSUBG = P // SUB
CPB = 40
TAIL = N - 39 * P
NCH = B * CPB
NTILES = 32
MAXC = (NCH + NTILES - 1) // NTILES


def _prep_kernel(locT_ref, idx_ref, msk_ref):
    x = locT_ref[0, 0]
    y = locT_ref[0, 1]
    m = (x > -1.0) & (x < 1.0) & (y > -1.0) & (y < 1.0)
    xs = jnp.clip(x, -0.999, 0.999) * AXES_LIMIT
    ys = jnp.clip(y, -0.999, 0.999) * AXES_LIMIT
    row = ((AXES_LIMIT - ys) / RESOLUTION).astype(jnp.int32)
    col = ((AXES_LIMIT + xs) / RESOLUTION).astype(jnp.int32)
    base = pl.program_id(0) * HW
    idx_ref[0, 0] = jnp.where(m, base + row * WL + col, -1)
    msk_ref[0, 0] = m.astype(jnp.int32)


_prep = pl.pallas_call(
    _prep_kernel,
    out_shape=(jax.ShapeDtypeStruct((B, 1, NPAD), jnp.int32),
               jax.ShapeDtypeStruct((B, 1, NPAD), jnp.int32)),
    grid=(B,),
    in_specs=[pl.BlockSpec((1, 2, NPAD), lambda b: (b, 0, 0))],
    out_specs=(pl.BlockSpec((1, 1, NPAD), lambda b: (b, 0, 0)),
               pl.BlockSpec((1, 1, NPAD), lambda b: (b, 0, 0))),
)


def _sc_body(table_hbm, idx_hbm, msk_hbm, empty_hbm, out_hbm,
             rowsbuf, idxbuf, emptybuf, mskbuf, sem):
    cid = lax.axis_index("c")
    sid = lax.axis_index("s")
    t = sid * 2 + cid

    pltpu.sync_copy(empty_hbm, emptybuf)
    evs = [emptybuf[pl.ds(16 * j, 16)] for j in range(8)]
    def do_chunk(b, k, npts, nsub):
        del nsub
        n0 = pl.multiple_of(k * P, SUB)
        pltpu.sync_copy(idx_hbm.at[b, 0, pl.ds(n0, npts)],
                        idxbuf.at[pl.ds(0, npts)])
        pltpu.sync_copy(msk_hbm.at[b, 0, pl.ds(n0, npts)],
                        mskbuf.at[pl.ds(0, npts)])
        copies = [
            pltpu.async_copy(
                table_hbm.at[plsc.Indices(idxbuf[pl.ds(16 * g, 16)],
                                          ignored_value=-1)],
                rowsbuf.at[pl.ds(16 * g, 16), :], sem)
            for g in range(npts // 16)
        ]
        for cp in copies:
            cp.wait()

        def fix_body(i, _):
            i16 = jnp.zeros((16,), jnp.int32) + i
            mv = plsc.load_gather(mskbuf, [i16])
            pred = mv != 0
            for jj in range(8):
                r = rowsbuf[i, pl.ds(16 * jj, 16)]
                rowsbuf[i, pl.ds(16 * jj, 16)] = jnp.where(pred, r, evs[jj])
            return 0

        lax.fori_loop(0, npts, fix_body, 0)
        out0 = pl.multiple_of(b * N + k * P, 8)
        pltpu.sync_copy(rowsbuf.at[pl.ds(0, npts), :],
                        out_hbm.at[pl.ds(out0, npts), :])

    for c in range(MAXC):
        j = t + NTILES * c

        @pl.when(j < NCH)
        def _chunk():
            b = j // CPB
            k = j % CPB

            @pl.when(k < CPB - 1)
            def _full():
                do_chunk(b, k, P, NSUBG)

            @pl.when(k == CPB - 1)
            def _tail():
                do_chunk(b, k, TAIL, 1)


@functools.partial(
    pl.kernel,
    out_type=jax.ShapeDtypeStruct((B * N, C), jnp.float32),
    mesh=plsc.VectorSubcoreMesh(core_axis_name="c", subcore_axis_name="s"),
    compiler_params=pltpu.CompilerParams(needs_layout_passes=False),
    scratch_types=[
        pltpu.VMEM((P, C), jnp.float32),
        pltpu.VMEM((P,), jnp.int32),
        pltpu.VMEM((C,), jnp.float32),
        pltpu.VMEM((P,), jnp.int32),
        pltpu.SemaphoreType.DMA,
    ],
)
def _sc_gather(table_hbm, idx_hbm, msk_hbm, empty_hbm, out_hbm, *scratch):
    _sc_body(table_hbm, idx_hbm, msk_hbm, empty_hbm, out_hbm, *scratch)


def kernel(fmap, loc, empty):
    table = fmap.transpose(0, 2, 3, 1).reshape(B * HW, C)
    locT = jnp.pad(loc.transpose(0, 2, 1), ((0, 0), (0, 0), (0, NPAD - N)),
                   constant_values=5.0)
    idx, msk = _prep(locT)
    out = _sc_gather(table, idx, msk, empty)
    return out.reshape(B, N, C)


# speedup vs baseline: 17.9449x; 1.1823x over previous
import functools

import jax
import jax.numpy as jnp
from jax import lax
from jax.experimental import pallas as pl
from jax.experimental.pallas import tpu as pltpu
from jax.experimental.pallas import tpu_sc as plsc

AXES_LIMIT = 40.0
RESOLUTION = 0.25
WL = int(AXES_LIMIT * 2 / RESOLUTION)

B = 2
C = 128
N = 20000
NPAD = 20480
HW = WL * WL

P = 256
CPBF = N // P
TAIL = N - CPBF * P
NCHF = B * CPBF
NTILES = 32
MAXC = (NCHF + NTILES - 1) // NTILES


def _prep_kernel(locT_ref, idx_ref, msk_ref):
    x = locT_ref[0, 0]
    y = locT_ref[0, 1]
    m = (x > -1.0) & (x < 1.0) & (y > -1.0) & (y < 1.0)
    xs = jnp.clip(x, -0.999, 0.999) * AXES_LIMIT
    ys = jnp.clip(y, -0.999, 0.999) * AXES_LIMIT
    row = ((AXES_LIMIT - ys) / RESOLUTION).astype(jnp.int32)
    col = ((AXES_LIMIT + xs) / RESOLUTION).astype(jnp.int32)
    base = pl.program_id(0) * HW
    idx_ref[0, 0] = jnp.where(m, base + row * WL + col, -1)
    msk_ref[0, 0] = m.astype(jnp.int32)


_prep = pl.pallas_call(
    _prep_kernel,
    out_shape=(jax.ShapeDtypeStruct((B, 1, NPAD), jnp.int32),
               jax.ShapeDtypeStruct((B, 1, NPAD), jnp.int32)),
    grid=(B,),
    in_specs=[pl.BlockSpec((1, 2, NPAD), lambda b: (b, 0, 0))],
    out_specs=(pl.BlockSpec((1, 1, NPAD), lambda b: (b, 0, 0)),
               pl.BlockSpec((1, 1, NPAD), lambda b: (b, 0, 0))),
)


def _sc_body(table_hbm, idx_hbm, msk_hbm, empty_hbm, out_hbm,
             rowsbuf, idxbuf, emptybuf, mskbuf, sem):
    cid = lax.axis_index("c")
    sid = lax.axis_index("s")
    t = sid * 2 + cid

    pltpu.sync_copy(empty_hbm, emptybuf)
    evs = [emptybuf[pl.ds(16 * j, 16)] for j in range(8)]

    def issue(b, n0, npts, half):
        hp = half * P
        pltpu.sync_copy(idx_hbm.at[b, 0, pl.ds(n0, npts)],
                        idxbuf.at[pl.ds(0, npts)])
        pltpu.sync_copy(msk_hbm.at[b, 0, pl.ds(n0, npts)],
                        mskbuf.at[pl.ds(hp, npts)])
        for g in range(npts // 16):
            pltpu.async_copy(
                table_hbm.at[plsc.Indices(idxbuf[pl.ds(16 * g, 16)],
                                          ignored_value=-1)],
                rowsbuf.at[pl.ds(hp + 16 * g, 16), :], sem)

    def finish(b, n0, npts, half):
        hp = half * P
        pltpu.make_async_copy(table_hbm.at[pl.ds(0, npts), :],
                              rowsbuf.at[pl.ds(hp, npts), :],
                              sem).wait()

        def fix_body(i, _):
            i16 = jnp.zeros((16,), jnp.int32) + hp + i
            mv = plsc.load_gather(mskbuf, [i16])
            pred = mv != 0
            for jj in range(8):
                r = rowsbuf[hp + i, pl.ds(16 * jj, 16)]
                rowsbuf[hp + i, pl.ds(16 * jj, 16)] = jnp.where(
                    pred, r, evs[jj])
            return 0

        lax.fori_loop(0, npts, fix_body, 0)
        out0 = pl.multiple_of(b * N, 8) + n0
        pltpu.sync_copy(rowsbuf.at[pl.ds(hp, npts), :],
                        out_hbm.at[pl.ds(out0, npts), :])

    def chunk_args(c):
        j = t + NTILES * c
        return j < NCHF, j // CPBF, pl.multiple_of((j % CPBF) * P, P)

    for c in range(MAXC):
        ok, b, n0 = chunk_args(c)

        @pl.when(ok)
        def _issue(b=b, n0=n0, c=c):
            issue(b, n0, P, c % 2)

        if c > 0:
            ok1, b1, n1 = chunk_args(c - 1)

            @pl.when(ok1)
            def _finish(b1=b1, n1=n1, c=c):
                finish(b1, n1, P, (c - 1) % 2)

    okl, bl, nl = chunk_args(MAXC - 1)

    @pl.when(okl)
    def _finish_last():
        finish(bl, nl, P, (MAXC - 1) % 2)

    @pl.when(t < B)
    def _tail():
        issue(t, CPBF * P, TAIL, 0)
        finish(t, CPBF * P, TAIL, 0)


@functools.partial(
    pl.kernel,
    out_type=jax.ShapeDtypeStruct((B * N, C), jnp.float32),
    mesh=plsc.VectorSubcoreMesh(core_axis_name="c", subcore_axis_name="s"),
    compiler_params=pltpu.CompilerParams(needs_layout_passes=False),
    scratch_types=[
        pltpu.VMEM((2 * P, C), jnp.float32),
        pltpu.VMEM((P,), jnp.int32),
        pltpu.VMEM((C,), jnp.float32),
        pltpu.VMEM((2 * P,), jnp.int32),
        pltpu.SemaphoreType.DMA,
    ],
)
def _sc_gather(table_hbm, idx_hbm, msk_hbm, empty_hbm, out_hbm, *scratch):
    _sc_body(table_hbm, idx_hbm, msk_hbm, empty_hbm, out_hbm, *scratch)


def kernel(fmap, loc, empty):
    table = fmap.transpose(0, 2, 3, 1).reshape(B * HW, C)
    locT = jnp.pad(loc.transpose(0, 2, 1), ((0, 0), (0, 0), (0, NPAD - N)),
                   constant_values=5.0)
    idx, msk = _prep(locT)
    out = _sc_gather(table, idx, msk, empty)
    return out.reshape(B, N, C)
